# baseline (speedup 1.0000x reference)
import functools

import jax
import jax.numpy as jnp
from jax import lax
from jax.experimental import pallas as pl
from jax.experimental.pallas import tpu as pltpu
from jax.experimental.pallas import tpu_sc as plsc

EMBED = 64
TOTAL = 2048
NUM_CORES = 2
NUM_SUBCORES = 16
DIMS_PER_CORE = EMBED // NUM_CORES
DIMS_PER_WORKER = DIMS_PER_CORE // NUM_SUBCORES
NROW = 16
NCOL = TOTAL // NROW
ENTITIES = 100000
CHUNK0 = 50048
CHUNK1 = ENTITIES - CHUNK0
TAIL = ENTITIES % 128


def _score_body(entT_hbm, relT_hbm, batchT_hbm, corruptT_hbm, out_hbm,
                hidx_v, ridx_v, ttidx_v, tidx_v,
                colA_v, colB_v, tail_v, diff_v, acc_v, shared_s,
                sem_i, sem_a, sem_b):
    c = lax.axis_index("c")
    s = lax.axis_index("s")
    d0 = c * DIMS_PER_CORE + s * DIMS_PER_WORKER

    half = TOTAL // 2
    idx_cps = [
        pltpu.async_copy(batchT_hbm.at[0], hidx_v.at[pl.ds(0, half)], sem_i),
        pltpu.async_copy(corruptT_hbm.at[0], hidx_v.at[pl.ds(half, half)], sem_i),
        pltpu.async_copy(batchT_hbm.at[1], ridx_v.at[pl.ds(0, half)], sem_i),
        pltpu.async_copy(corruptT_hbm.at[1], ridx_v.at[pl.ds(half, half)], sem_i),
        pltpu.async_copy(batchT_hbm.at[3], ttidx_v.at[pl.ds(0, half)], sem_i),
        pltpu.async_copy(corruptT_hbm.at[3], ttidx_v.at[pl.ds(half, half)], sem_i),
        pltpu.async_copy(batchT_hbm.at[2], tidx_v.at[pl.ds(0, half)], sem_i),
        pltpu.async_copy(corruptT_hbm.at[2], tidx_v.at[pl.ds(half, half)], sem_i),
    ]

    bufs = (colA_v, colB_v)
    sems = (sem_a, sem_b)

    def fire(i):
        k, phase, hh = i // 4, (i // 2) % 2, i % 2
        tbl = entT_hbm if phase == 0 else relT_hbm
        d = d0 + k
        dst = bufs[i % 2]
        sem = sems[i % 2]
        if hh == 0:
            return [pltpu.async_copy(tbl.at[d, pl.ds(0, CHUNK0)],
                                     dst.at[pl.ds(0, CHUNK0)], sem)]
        bulk = CHUNK1 - TAIL
        return [
            pltpu.async_copy(tbl.at[d, pl.ds(CHUNK0, bulk)],
                             dst.at[pl.ds(0, bulk)], sem),
            pltpu.async_copy(tbl.at[d, pl.ds(CHUNK0 + bulk, TAIL)],
                             tail_v, sem),
        ]

    cps = {0: fire(0), 1: fire(1)}
    for cp in cps[0] + cps[1]:
        cp.wait()
    for cp in idx_cps:
        cp.wait()

    def masked_pair(col, idx_a, idx_b, base, lim):
        la = idx_a - base
        lb = idx_b - base
        ma = plsc.bitcast(la, jnp.uint32) < lim
        mb = plsc.bitcast(lb, jnp.uint32) < lim
        ga = plsc.load_gather(col, [la], mask=ma)
        gb = plsc.load_gather(col, [lb], mask=mb)
        return ga, gb

    for i in range(8):
        k, phase, hh = i // 4, (i // 2) % 2, i % 2
        col = bufs[i % 2]
        base = jnp.int32(0 if hh == 0 else CHUNK0)
        lim = jnp.uint32(CHUNK0 if hh == 0 else CHUNK1)
        if i >= 2:
            for cp in cps[i]:
                cp.wait()
        if hh == 1:
            bulk = CHUNK1 - TAIL
            col[pl.ds(bulk, 16)] = tail_v[pl.ds(0, 16)]
            col[pl.ds(bulk + 16, 16)] = tail_v[pl.ds(16, 16)]

        if phase == 0:
            def ent_row(row, _):
                for j in range(NCOL // 16):
                    b = row * NCOL + j * 16
                    eh, et = masked_pair(col, hidx_v[pl.ds(b, 16)],
                                         tidx_v[pl.ds(b, 16)], base, lim)
                    sl = pl.ds(j * 16, 16)
                    if hh == 0:
                        diff_v[row, sl] = eh - et
                    else:
                        diff_v[row, sl] = diff_v[row, sl] + eh - et
                return 0

            lax.fori_loop(0, NROW, ent_row, 0)
        else:
            def rel_row(row, _):
                for j in range(NCOL // 16):
                    b = row * NCOL + j * 16
                    rr, rtt = masked_pair(col, ridx_v[pl.ds(b, 16)],
                                          ttidx_v[pl.ds(b, 16)], base, lim)
                    sl = pl.ds(j * 16, 16)
                    if hh == 0:
                        diff_v[row, sl] = diff_v[row, sl] + rr + rtt
                    else:
                        e = diff_v[row, sl] + rr + rtt
                        if k == 0:
                            acc_v[row, sl] = e * e
                        else:
                            acc_v[row, sl] = acc_v[row, sl] + e * e
                return 0

            lax.fori_loop(0, NROW, rel_row, 0)

        if i + 2 < 8:
            cps[i + 2] = fire(i + 2)

    rows = lax.iota(jnp.int32, 16)

    @pl.when(s == 0)
    def _():
        pltpu.sync_copy(acc_v, shared_s)

    plsc.subcore_barrier()

    @pl.when(s != 0)
    def _():
        pltpu.sync_copy(acc_v, shared_s.at[rows], add=True)

    plsc.subcore_barrier()

    @pl.when(s == 0)
    def _():
        pltpu.sync_copy(shared_s, out_hbm.at[c])


@jax.jit
def _ttranse_scores(entT, relT, batchT, corruptT):
    call = functools.partial(
        pl.kernel,
        out_type=jax.ShapeDtypeStruct((NUM_CORES, NROW, NCOL), jnp.float32),
        mesh=plsc.VectorSubcoreMesh(core_axis_name="c", subcore_axis_name="s"),
        compiler_params=pltpu.CompilerParams(
            needs_layout_passes=False, use_tc_tiling_on_sc=True),
        scratch_types=[
            pltpu.VMEM((TOTAL,), jnp.int32),
            pltpu.VMEM((TOTAL,), jnp.int32),
            pltpu.VMEM((TOTAL,), jnp.int32),
            pltpu.VMEM((TOTAL,), jnp.int32),
            pltpu.VMEM((CHUNK0,), jnp.float32),
            pltpu.VMEM((CHUNK0,), jnp.float32),
            pltpu.VMEM((TAIL,), jnp.float32),
            pltpu.VMEM((NROW, NCOL), jnp.float32),
            pltpu.VMEM((NROW, NCOL), jnp.float32),
            pltpu.VMEM_SHARED((NROW, NCOL), jnp.float32),
            pltpu.SemaphoreType.DMA,
            pltpu.SemaphoreType.DMA,
            pltpu.SemaphoreType.DMA,
        ],
    )(_score_body)
    return call(entT, relT, batchT, corruptT)


def kernel(batch, corrupt_batch, entity_embedding, relation_embedding):
    out = _ttranse_scores(entity_embedding.T, relation_embedding.T,
                          batch.T.astype(jnp.int32),
                          corrupt_batch.T.astype(jnp.int32))
    total = (out[0] + out[1]).reshape(TOTAL)
    n = batch.shape[0]
    return (total[:n], total[n:])


# speedup vs baseline: 1.0857x; 1.0857x over previous
import functools

import jax
import jax.numpy as jnp
from jax import lax
from jax.experimental import pallas as pl
from jax.experimental.pallas import tpu as pltpu
from jax.experimental.pallas import tpu_sc as plsc

EMBED = 64
TOTAL = 2048
NUM_CORES = 2
NUM_SUBCORES = 16
DIMS_PER_CORE = EMBED // NUM_CORES
DIMS_PER_WORKER = DIMS_PER_CORE // NUM_SUBCORES
NROW = 16
NCOL = TOTAL // NROW
ENTITIES = 100000


def _score_body(entT_hbm, relT_hbm, batchT_hbm, corruptT_hbm, out_hbm,
                hidx_v, ridx_v, ttidx_v, tidx_v,
                col_v, diff_v, acc_v, shared_s, sem_i, sem_c):
    c = lax.axis_index("c")
    s = lax.axis_index("s")
    d0 = c * DIMS_PER_CORE + s * DIMS_PER_WORKER

    half = TOTAL // 2
    idx_cps = [
        pltpu.async_copy(batchT_hbm.at[0], hidx_v.at[pl.ds(0, half)], sem_i),
        pltpu.async_copy(corruptT_hbm.at[0], hidx_v.at[pl.ds(half, half)], sem_i),
        pltpu.async_copy(batchT_hbm.at[1], ridx_v.at[pl.ds(0, half)], sem_i),
        pltpu.async_copy(corruptT_hbm.at[1], ridx_v.at[pl.ds(half, half)], sem_i),
        pltpu.async_copy(batchT_hbm.at[3], ttidx_v.at[pl.ds(0, half)], sem_i),
        pltpu.async_copy(corruptT_hbm.at[3], ttidx_v.at[pl.ds(half, half)], sem_i),
        pltpu.async_copy(batchT_hbm.at[2], tidx_v.at[pl.ds(0, half)], sem_i),
        pltpu.async_copy(corruptT_hbm.at[2], tidx_v.at[pl.ds(half, half)], sem_i),
    ]
    col_cp = pltpu.async_copy(entT_hbm.at[d0], col_v, sem_c)
    for cp in idx_cps:
        cp.wait()

    for k in range(DIMS_PER_WORKER):
        d = d0 + k

        if k == 0:
            col_cp.wait()
        else:
            pltpu.sync_copy(entT_hbm.at[d], col_v)

        def ent_row(row, _):
            for j in range(NCOL // 16):
                base = row * NCOL + j * 16
                hi = hidx_v[pl.ds(base, 16)]
                ti = tidx_v[pl.ds(base, 16)]
                eh = plsc.load_gather(col_v, [hi])
                et = plsc.load_gather(col_v, [ti])
                diff_v[row, pl.ds(j * 16, 16)] = eh - et
            return 0

        lax.fori_loop(0, NROW, ent_row, 0)

        pltpu.sync_copy(relT_hbm.at[d], col_v)

        def rel_row(row, _):
            for j in range(NCOL // 16):
                base = row * NCOL + j * 16
                ri = ridx_v[pl.ds(base, 16)]
                tti = ttidx_v[pl.ds(base, 16)]
                rr = plsc.load_gather(col_v, [ri])
                rtt = plsc.load_gather(col_v, [tti])
                sl = pl.ds(j * 16, 16)
                e = diff_v[row, sl] + rr + rtt
                if k == 0:
                    acc_v[row, sl] = e * e
                else:
                    acc_v[row, sl] = acc_v[row, sl] + e * e
            return 0

        lax.fori_loop(0, NROW, rel_row, 0)

    rows = lax.iota(jnp.int32, 16)

    @pl.when(s == 0)
    def _():
        pltpu.sync_copy(acc_v, shared_s)

    plsc.subcore_barrier()

    @pl.when(s != 0)
    def _():
        pltpu.sync_copy(acc_v, shared_s.at[rows], add=True)

    plsc.subcore_barrier()

    @pl.when(s == 0)
    def _():
        pltpu.sync_copy(shared_s, out_hbm.at[c])


@jax.jit
def _ttranse_scores(entT, relT, batchT, corruptT):
    call = functools.partial(
        pl.kernel,
        out_type=jax.ShapeDtypeStruct((NUM_CORES, NROW, NCOL), jnp.float32),
        mesh=plsc.VectorSubcoreMesh(core_axis_name="c", subcore_axis_name="s"),
        compiler_params=pltpu.CompilerParams(
            needs_layout_passes=False, use_tc_tiling_on_sc=True),
        scratch_types=[
            pltpu.VMEM((TOTAL,), jnp.int32),
            pltpu.VMEM((TOTAL,), jnp.int32),
            pltpu.VMEM((TOTAL,), jnp.int32),
            pltpu.VMEM((TOTAL,), jnp.int32),
            pltpu.VMEM((ENTITIES,), jnp.float32),
            pltpu.VMEM((NROW, NCOL), jnp.float32),
            pltpu.VMEM((NROW, NCOL), jnp.float32),
            pltpu.VMEM_SHARED((NROW, NCOL), jnp.float32),
            pltpu.SemaphoreType.DMA,
            pltpu.SemaphoreType.DMA,
        ],
    )(_score_body)
    return call(entT, relT, batchT, corruptT)


def kernel(batch, corrupt_batch, entity_embedding, relation_embedding):
    out = _ttranse_scores(entity_embedding.T, relation_embedding.T,
                          batch.T.astype(jnp.int32),
                          corrupt_batch.T.astype(jnp.int32))
    hr = NROW // 2
    correct = (out[0, :hr] + out[1, :hr]).reshape(TOTAL // 2)
    corrupt = (out[0, hr:] + out[1, hr:]).reshape(TOTAL // 2)
    return (correct, corrupt)
